# baseline (speedup 1.0000x reference)
import functools

import jax
import jax.numpy as jnp
from jax import lax
from jax.experimental import pallas as pl
from jax.experimental.pallas import tpu as pltpu
from jax.experimental.pallas import tpu_sc as plsc

T = 2048
D = 768
F = 1536
E = 16
TOPK = 4
A = T * TOPK
TILE = 128
NTILES = A // TILE + E - 1
NTILES = 80
NPAD = NTILES * TILE
NW = 32


def _route_body(x_ref, gw_ref, p4_ref, w4_ref, te_ref):
    xf = x_ref[...]
    logits = lax.dot_general(xf, gw_ref[...], (((1,), (1,)), ((), ())),
                             preferred_element_type=jnp.float32)
    m = jnp.max(logits, axis=1, keepdims=True)
    p = jnp.exp(logits - m)
    p = p / jnp.sum(p, axis=1, keepdims=True)

    lane = lax.broadcasted_iota(jnp.int32, (T, E), 1)
    work = p
    onehots = []
    for _ in range(TOPK):
        mx = jnp.max(work, axis=1, keepdims=True)
        cand = jnp.where(work == mx, lane, E)
        first = jnp.min(cand, axis=1, keepdims=True)
        oh = (lane == first).astype(jnp.float32)
        onehots.append(oh)
        work = jnp.where(oh > 0, -1.0, work)
    mask = onehots[0] + onehots[1] + onehots[2] + onehots[3]

    c = mask
    s = 1
    while s < T:
        c = c + jnp.concatenate(
            [jnp.zeros((s, E), jnp.float32), lax.slice(c, (0, 0), (T - s, E))],
            axis=0)
        s *= 2
    rank = c - mask

    counts_row = lax.slice(c, (T - 1, 0), (T, E))
    tiles_row = jnp.floor((counts_row + (TILE - 1)) * (1.0 / TILE))
    ii = lax.broadcasted_iota(jnp.int32, (E, E), 0)
    jj = lax.broadcasted_iota(jnp.int32, (E, E), 1)
    tri_row = (ii < jj).astype(jnp.float32)
    start_row = lax.dot_general(tiles_row, tri_row, (((1,), (0,)), ((), ())),
                                preferred_element_type=jnp.float32)
    offs_row = start_row * float(TILE)
    pos = offs_row + rank

    p4cols = [jnp.sum(oh * pos, axis=1, keepdims=True) for oh in onehots]
    w4cols = [jnp.sum(oh * p, axis=1, keepdims=True) for oh in onehots]
    p4_ref[...] = jnp.concatenate(p4cols, axis=1).astype(jnp.int32)
    ones16 = jnp.ones((1, 16), jnp.float32)
    w4_ref[...] = jnp.concatenate([wc * ones16 for wc in w4cols], axis=1)

    ones_col = jnp.ones((T, 1), jnp.float32)
    counts_col = lax.dot_general(mask, ones_col, (((0,), (0,)), ((), ())),
                                 preferred_element_type=jnp.float32)
    tiles_col = jnp.floor((counts_col + (TILE - 1)) * (1.0 / TILE))
    tri_col = (jj < ii).astype(jnp.float32)
    start_col = lax.dot_general(tri_col, tiles_col, (((1,), (0,)), ((), ())),
                                preferred_element_type=jnp.float32)
    end_col = start_col + tiles_col
    ti = lax.broadcasted_iota(jnp.int32, (E, NTILES), 1).astype(jnp.float32)
    te = jnp.sum((ti >= end_col).astype(jnp.float32), axis=0, keepdims=True)
    te_ref[...] = jnp.minimum(te, float(E - 1)).astype(jnp.int32)


def _route(x, gate_w):
    return pl.pallas_call(
        _route_body,
        grid=(1,),
        in_specs=[
            pl.BlockSpec((T, D), lambda i: (0, 0)),
            pl.BlockSpec((E, D), lambda i: (0, 0)),
        ],
        out_specs=[
            pl.BlockSpec((T, TOPK), lambda i: (0, 0)),
            pl.BlockSpec((T, TOPK * 16), lambda i: (0, 0)),
            pl.BlockSpec((1, NTILES), lambda i: (0, 0)),
        ],
        out_shape=[
            jax.ShapeDtypeStruct((T, TOPK), jnp.int32),
            jax.ShapeDtypeStruct((T, TOPK * 16), jnp.float32),
            jax.ShapeDtypeStruct((1, NTILES), jnp.int32),
        ],
    )(x, gate_w)


_APW = A // NW


def _scatter_body(p4_hbm, tok_hbm, pv, tokv_v, sem1, sem2):
    wid = lax.axis_index("s") * 2 + lax.axis_index("c")
    base = wid * _APW
    pltpu.sync_copy(p4_hbm.at[pl.ds(base, 128)], pv.at[0])
    pltpu.sync_copy(p4_hbm.at[pl.ds(base + 128, 128)], pv.at[1])
    iota = lax.broadcasted_iota(jnp.int32, (16,), 0)
    for j in range(_APW // 16):
        tokv_v[pl.ds(j * 16, 16)] = lax.shift_right_logical(
            base + j * 16 + iota, 2)
    cp1 = pltpu.async_copy(tokv_v.at[pl.ds(0, 128)], tok_hbm.at[pv.at[0]], sem1)
    cp2 = pltpu.async_copy(tokv_v.at[pl.ds(128, 128)], tok_hbm.at[pv.at[1]], sem2)
    cp1.wait()
    cp2.wait()


_RPW = NPAD // NW
_GCH = 64


def _gather_body(tok_hbm, x_hbm, xg_hbm, idx2, rowsv, sem):
    wid = lax.axis_index("s") * 2 + lax.axis_index("c")
    rb0 = wid * _RPW
    for ci in range(_RPW // _GCH):
        rb = rb0 + ci * _GCH
        pltpu.sync_copy(tok_hbm.at[pl.ds(rb, _GCH)], idx2.at[ci])
        for j in range(_GCH // 16):
            v = idx2[ci, pl.ds(j * 16, 16)]
            idx2[ci, pl.ds(j * 16, 16)] = jnp.minimum(
                jnp.maximum(v, 0), T - 1)
        pltpu.async_copy(x_hbm.at[idx2.at[ci]], rowsv, sem).wait()
        pltpu.sync_copy(rowsv, xg_hbm.at[pl.ds(rb, _GCH)])


def _gmm_body(te_ref, xg_ref, w1_ref, b1_ref, w2_ref, b2_ref, ys_ref):
    xb = xg_ref[...].astype(jnp.bfloat16)
    w1 = w1_ref[0].astype(jnp.bfloat16)
    w2 = w2_ref[0].astype(jnp.bfloat16)
    h = lax.dot_general(xb, w1, (((1,), (1,)), ((), ())),
                        preferred_element_type=jnp.float32)
    h = jnp.maximum(h + b1_ref[0], 0.0).astype(jnp.bfloat16)
    y = lax.dot_general(h, w2, (((1,), (1,)), ((), ())),
                        preferred_element_type=jnp.float32)
    ys_ref[...] = y + b2_ref[0]


def _gmm(te, xg, W1, b1, W2, b2):
    grid_spec = pltpu.PrefetchScalarGridSpec(
        num_scalar_prefetch=1,
        grid=(NTILES,),
        in_specs=[
            pl.BlockSpec((TILE, D), lambda i, te: (i, 0)),
            pl.BlockSpec((1, F, D), lambda i, te: (te[0, i], 0, 0)),
            pl.BlockSpec((1, 1, F), lambda i, te: (te[0, i], 0, 0)),
            pl.BlockSpec((1, D, F), lambda i, te: (te[0, i], 0, 0)),
            pl.BlockSpec((1, 1, D), lambda i, te: (te[0, i], 0, 0)),
        ],
        out_specs=pl.BlockSpec((TILE, D), lambda i, te: (i, 0)),
    )
    return pl.pallas_call(
        _gmm_body,
        grid_spec=grid_spec,
        out_shape=jax.ShapeDtypeStruct((NPAD, D), jnp.float32),
    )(te, xg, W1, b1.reshape(E, 1, F), W2, b2.reshape(E, 1, D))


_TPW = T // NW
_CCH = 16


def _combine_body(p4_hbm, w4b_hbm, ys_hbm, x_hbm, out_hbm,
                  pvr, wvr, yr, xr, outv, sem):
    wid = lax.axis_index("s") * 2 + lax.axis_index("c")
    tb0 = wid * _TPW
    for ci in range(_TPW // _CCH):
        tb = tb0 + ci * _CCH
        pltpu.sync_copy(p4_hbm.at[pl.ds(tb * TOPK, _CCH * TOPK)], pvr.at[ci])
        pltpu.async_copy(ys_hbm.at[pvr.at[ci]], yr, sem).wait()
        pltpu.sync_copy(x_hbm.at[pl.ds(tb, _CCH)], xr)
        pltpu.sync_copy(w4b_hbm.at[pl.ds(tb, _CCH)], wvr)

        def body(j, carry):
            w0 = wvr[j, pl.ds(0, 16)]
            w1 = wvr[j, pl.ds(16, 16)]
            w2 = wvr[j, pl.ds(32, 16)]
            w3 = wvr[j, pl.ds(48, 16)]
            for v in range(D // 16):
                sl = pl.ds(v * 16, 16)
                acc = (xr[j, sl] + w0 * yr[4 * j, sl] + w1 * yr[4 * j + 1, sl]
                       + w2 * yr[4 * j + 2, sl] + w3 * yr[4 * j + 3, sl])
                outv[j, sl] = acc
            return carry

        lax.fori_loop(0, _CCH, body, 0)
        pltpu.sync_copy(outv, out_hbm.at[pl.ds(tb, _CCH)])


@functools.cache
def _sc_kernels():
    mesh = plsc.VectorSubcoreMesh(core_axis_name="c", subcore_axis_name="s",
                                  num_cores=2, num_subcores=16)
    scatter_k = pl.kernel(
        _scatter_body,
        out_type=jax.ShapeDtypeStruct((NPAD,), jnp.int32),
        mesh=mesh,
        scratch_types=[
            pltpu.VMEM((2, 128), jnp.int32),
            pltpu.VMEM((_APW,), jnp.int32),
            pltpu.SemaphoreType.DMA,
            pltpu.SemaphoreType.DMA,
        ],
    )
    gather_k = pl.kernel(
        _gather_body,
        out_type=jax.ShapeDtypeStruct((NPAD, D), jnp.float32),
        mesh=mesh,
        scratch_types=[
            pltpu.VMEM((_RPW // _GCH, _GCH), jnp.int32),
            pltpu.VMEM((_GCH, D), jnp.float32),
            pltpu.SemaphoreType.DMA,
        ],
    )
    combine_k = pl.kernel(
        _combine_body,
        out_type=jax.ShapeDtypeStruct((T, D), jnp.float32),
        mesh=mesh,
        scratch_types=[
            pltpu.VMEM((_TPW // _CCH, _CCH * TOPK), jnp.int32),
            pltpu.VMEM((_CCH, TOPK * 16), jnp.float32),
            pltpu.VMEM((_CCH * TOPK, D), jnp.float32),
            pltpu.VMEM((_CCH, D), jnp.float32),
            pltpu.VMEM((_CCH, D), jnp.float32),
            pltpu.SemaphoreType.DMA,
        ],
    )
    return scatter_k, gather_k, combine_k


def kernel(x, gate_w, W1, b1, W2, b2):
    scatter_k, gather_k, combine_k = _sc_kernels()
    p4, w4b, te = _route(x, gate_w)
    p4f = p4.reshape(A)
    tok = scatter_k(p4f)
    xg = gather_k(tok, x)
    ys = _gmm(te, xg, W1, b1, W2, b2)
    return combine_k(p4f, w4b, ys, x)


# speedup vs baseline: 2.2953x; 2.2953x over previous
import jax
import jax.numpy as jnp
from jax.experimental import pallas as pl
from jax.experimental.pallas import tpu as pltpu

TOKENS = 2048
D = 768
F = 1536
E = 16
TOPK = 4
TBLK = 1024


def _moe_body(x_ref, gw_ref, w1_ref, b1_ref, w2_ref, b2_ref, out_ref,
              probs_ref, xbf_ref):
    e = pl.program_id(0)

    @pl.when(e == 0)
    def _gating():
        xf = x_ref[...]
        logits = jax.lax.dot_general(
            xf, gw_ref[...], (((1,), (1,)), ((), ())),
            preferred_element_type=jnp.float32)
        m = jnp.max(logits, axis=1, keepdims=True)
        p = jnp.exp(logits - m)
        p = p / jnp.sum(p, axis=1, keepdims=True)
        lane = jax.lax.broadcasted_iota(jnp.int32, (TOKENS, E), 1)
        work = p
        sel = jnp.zeros((TOKENS, E), jnp.float32)
        for _ in range(TOPK):
            mx = jnp.max(work, axis=1, keepdims=True)
            cand = jnp.where(work == mx, lane, E)
            first = jnp.min(cand, axis=1, keepdims=True)
            onehot = lane == first
            sel = jnp.where(onehot, 1.0, sel)
            work = jnp.where(onehot, -1.0, work)
        probs_ref[...] = p * sel
        out_ref[...] = xf
        xbf_ref[...] = xf.astype(jnp.bfloat16)

    lane = jax.lax.broadcasted_iota(jnp.int32, (TOKENS, E), 1)
    wcol = jnp.sum(probs_ref[...] * jnp.where(lane == e, 1.0, 0.0),
                   axis=1, keepdims=True)
    w1 = w1_ref[0].astype(jnp.bfloat16)
    w2 = w2_ref[0].astype(jnp.bfloat16)
    b1v = b1_ref[0]
    b2v = b2_ref[0]
    for j in range(TOKENS // TBLK):
        xb = xbf_ref[pl.ds(j * TBLK, TBLK), :]
        h = jax.lax.dot_general(xb, w1, (((1,), (1,)), ((), ())),
                                preferred_element_type=jnp.float32)
        h = jnp.maximum((h + b1v).astype(jnp.bfloat16), 0)
        y = jax.lax.dot_general(h, w2, (((1,), (1,)), ((), ())),
                                preferred_element_type=jnp.float32)
        y = y + b2v
        wj = jax.lax.slice(wcol, (j * TBLK, 0), ((j + 1) * TBLK, 1))
        out_ref[pl.ds(j * TBLK, TBLK), :] += wj * y


def kernel(x, gate_w, W1, b1, W2, b2):
    return pl.pallas_call(
        _moe_body,
        grid=(E,),
        in_specs=[
            pl.BlockSpec((TOKENS, D), lambda e: (0, 0)),
            pl.BlockSpec((E, D), lambda e: (0, 0)),
            pl.BlockSpec((1, F, D), lambda e: (e, 0, 0)),
            pl.BlockSpec((1, 1, F), lambda e: (e, 0, 0)),
            pl.BlockSpec((1, D, F), lambda e: (e, 0, 0)),
            pl.BlockSpec((1, 1, D), lambda e: (e, 0, 0)),
        ],
        out_specs=pl.BlockSpec((TOKENS, D), lambda e: (0, 0)),
        out_shape=jax.ShapeDtypeStruct((TOKENS, D), jnp.float32),
        scratch_shapes=[pltpu.VMEM((TOKENS, E), jnp.float32),
                        pltpu.VMEM((TOKENS, D), jnp.bfloat16)],
    )(x, gate_w, W1, b1.reshape(E, 1, F), W2, b2.reshape(E, 1, D))
